# baseline (speedup 1.0000x reference)
import dataclasses
import functools

import jax
import jax.numpy as jnp
from jax import lax
from jax.experimental import pallas as pl
from jax.experimental.pallas import tpu as pltpu
from jax.experimental.pallas import tpu_sc as plsc

N = 10000
E = 320000
D = 128
H1 = 64
H2 = 32

NC = 2
NS = 16
NW = NC * NS
K = 128

NPAD = 10240
ER = 2560
EPT = ER // NW
APT = NPAD // NS
IB = 16


def _sc_degree(dst2d):
    mesh = plsc.VectorSubcoreMesh(core_axis_name="c", subcore_axis_name="s")

    hr = NPAD // 128

    cp = pltpu.CompilerParams()
    if "needs_layout_passes" in pltpu.CompilerParams.__dataclass_fields__:
        cp = dataclasses.replace(cp, needs_layout_passes=False)

    @functools.partial(
        pl.kernel,
        out_type=jax.ShapeDtypeStruct((NC * hr, 128), jnp.float32),
        mesh=mesh,
        compiler_params=cp,
        scratch_types=[
            pltpu.VMEM((EPT, K), jnp.int32),
            pltpu.VMEM((hr, 128), jnp.float32),
            pltpu.VMEM((hr // 16, 16), jnp.int32),
            pltpu.VMEM_SHARED((hr, 128), jnp.float32),
            pltpu.SemaphoreType.DMA,
        ],
    )
    def deg_kernel(dst_hbm, out_hbm, idx_v, hist_v, idr_v, acc, sem):
        cid = lax.axis_index("c")
        sid = lax.axis_index("s")
        wid = sid * NC + cid

        pltpu.async_copy(
            dst_hbm.at[pl.ds(pl.multiple_of(wid * EPT, 8), EPT)], idx_v, sem)

        @pl.loop(0, hr)
        def _(r):
            @pl.loop(0, 128, step=16)
            def _(c):
                hist_v[r, pl.ds(c, 16)] = jnp.zeros((16,), jnp.float32)

        @pl.loop(0, hr // 16)
        def _(k):
            idr_v[k, :] = lax.iota(jnp.int32, 16) + k * 16

        @pl.when(sid < hr // 8)
        def _():
            pltpu.sync_copy(
                hist_v.at[pl.ds(0, 8)],
                acc.at[pl.ds(pl.multiple_of(sid * 8, 8), 8)])

        pltpu.make_async_copy(
            dst_hbm.at[pl.ds(pl.multiple_of(wid * EPT, 8), EPT)], idx_v,
            sem).wait()
        plsc.subcore_barrier()

        ones16 = jnp.ones((16,), jnp.float32)

        @pl.loop(0, EPT)
        def _(r):
            @pl.loop(0, K, step=16)
            def _(c):
                node = idx_v[r, pl.ds(c, 16)]
                plsc.addupdate_scatter(
                    hist_v,
                    [lax.shift_right_logical(node, 7),
                     lax.bitwise_and(node, 127)],
                    ones16)

        @pl.loop(0, hr // 16)
        def _(k):
            pltpu.sync_copy(
                hist_v.at[pl.ds(pl.multiple_of(k * 16, 8), 16)],
                acc.at[idr_v.at[k]], add=True)

        plsc.subcore_barrier()

        @pl.when(sid < hr // 8)
        def _():
            pltpu.sync_copy(
                acc.at[pl.ds(pl.multiple_of(sid * 8, 8), 8)],
                out_hbm.at[pl.ds(pl.multiple_of(cid * hr + sid * 8, 8), 8)])

    return deg_kernel(dst2d)


def _sc_aggregate(u, src2d, dst2d):
    mesh = plsc.VectorSubcoreMesh(core_axis_name="c", subcore_axis_name="s")

    @functools.partial(
        pl.kernel,
        out_type=jax.ShapeDtypeStruct((NC * NPAD, 128), jnp.float32),
        mesh=mesh,
        scratch_types=[
            pltpu.VMEM((IB, K), jnp.int32),
            pltpu.VMEM((IB, K), jnp.int32),
            pltpu.VMEM((K, 128), jnp.float32),
            pltpu.VMEM((K, 128), jnp.float32),
            pltpu.VMEM((8, 128), jnp.float32),
            pltpu.VMEM_SHARED((NPAD, 128), jnp.float32),
            pltpu.SemaphoreType.DMA,
            pltpu.SemaphoreType.DMA,
            pltpu.SemaphoreType.DMA,
            pltpu.SemaphoreType.DMA,
            pltpu.SemaphoreType.DMA,
        ],
    )
    def agg_kernel(u_hbm, src_hbm, dst_hbm, out_hbm,
                   src_v, dst_v, buf_a, buf_b, zero_v, acc,
                   sem_i, sem_a0, sem_a1, sem_b0, sem_b1):
        cid = lax.axis_index("c")
        sid = lax.axis_index("s")
        wid = sid * NC + cid

        @pl.loop(0, 8)
        def _(r):
            @pl.loop(0, 128, step=16)
            def _(c):
                zero_v[r, pl.ds(c, 16)] = jnp.zeros((16,), jnp.float32)

        base = pl.multiple_of(sid * APT, 8)

        @pl.loop(0, APT // 8)
        def _(j):
            pltpu.sync_copy(zero_v, acc.at[pl.ds(pl.multiple_of(base + j * 8, 8), 8)])

        ebase = wid * EPT
        plsc.subcore_barrier()

        HB = K // 2

        def g(bufref, half, rows_slice, sem):
            pltpu.async_copy(u_hbm.at[rows_slice],
                             bufref.at[pl.ds(half * HB, HB)], sem)

        def gw(bufref, half, rows_slice, sem):
            pltpu.make_async_copy(u_hbm.at[rows_slice],
                                  bufref.at[pl.ds(half * HB, HB)], sem).wait()

        @pl.loop(0, EPT // IB)
        def _(blk):
            off = pl.multiple_of(ebase + blk * IB, 8)
            pltpu.async_copy(src_hbm.at[pl.ds(off, IB)], src_v, sem_i).wait()
            pltpu.async_copy(dst_hbm.at[pl.ds(off, IB)], dst_v, sem_i).wait()

            def half_idx(j, h):
                return src_v.at[j, pl.ds(h * HB, HB)]

            g(buf_a, 0, half_idx(0, 0), sem_a0)
            g(buf_a, 1, half_idx(0, 1), sem_a1)
            g(buf_b, 0, half_idx(1, 0), sem_b0)
            g(buf_b, 1, half_idx(1, 1), sem_b1)

            @pl.loop(0, IB, step=2)
            def _(j):
                gw(buf_a, 0, half_idx(j, 0), sem_a0)
                gw(buf_a, 1, half_idx(j, 1), sem_a1)
                pltpu.sync_copy(buf_a, acc.at[dst_v.at[j]], add=True)

                @pl.when(j + 2 < IB)
                def _():
                    g(buf_a, 0, half_idx(j + 2, 0), sem_a0)
                    g(buf_a, 1, half_idx(j + 2, 1), sem_a1)

                gw(buf_b, 0, half_idx(j + 1, 0), sem_b0)
                gw(buf_b, 1, half_idx(j + 1, 1), sem_b1)
                pltpu.sync_copy(buf_b, acc.at[dst_v.at[j + 1]], add=True)

                @pl.when(j + 3 < IB)
                def _():
                    g(buf_b, 0, half_idx(j + 3, 0), sem_b0)
                    g(buf_b, 1, half_idx(j + 3, 1), sem_b1)

        plsc.subcore_barrier()
        pltpu.sync_copy(acc.at[pl.ds(base, APT)],
                        out_hbm.at[pl.ds(pl.multiple_of(cid * NPAD + base, 8), APT)])

    return agg_kernel(u, src2d, dst2d)


_DOT = functools.partial(
    lax.dot_general,
    dimension_numbers=(((1,), (0,)), ((), ())),
    preferred_element_type=jnp.float32,
    precision=lax.Precision.HIGHEST,
)

BM = 1024


def _dis(d0_ref, d1_ref):
    return lax.rsqrt(d0_ref[...] + d1_ref[...] + 1.0)


def _deg_spec(nb):
    return [pl.BlockSpec((BM, 1), lambda i: (i, 0)),
            pl.BlockSpec((BM, 1), lambda i, _nb=nb: (i + _nb, 0))]


def _scale_body(d0_ref, d1_ref, x_ref, u_ref):
    u_ref[...] = _dis(d0_ref, d1_ref) * x_ref[...]


def _tc_scale(deg2, x_pad):
    nb = NPAD // BM
    return pl.pallas_call(
        _scale_body,
        grid=(nb,),
        in_specs=_deg_spec(nb) + [pl.BlockSpec((BM, D), lambda i: (i, 0))],
        out_specs=pl.BlockSpec((BM, D), lambda i: (i, 0)),
        out_shape=jax.ShapeDtypeStruct((NPAD, D), jnp.float32),
    )(deg2, deg2, x_pad)


def _layer1_body(d0_ref, d1_ref, p0_ref, p1_ref, u_ref, w_ref, b_ref, v_ref):
    dis = _dis(d0_ref, d1_ref)
    z = p0_ref[...] + p1_ref[...] + u_ref[...]
    o = jnp.maximum(dis * _DOT(z, w_ref[...]) + b_ref[...], 0.0)
    v = dis * o
    v_ref[...] = jnp.concatenate(
        [v, jnp.zeros((v.shape[0], 128 - H1), jnp.float32)], axis=1)


def _tc_layer1(deg2, parts, u, W1, b1_row):
    nb = NPAD // BM
    return pl.pallas_call(
        _layer1_body,
        grid=(nb,),
        in_specs=_deg_spec(nb) + [
            pl.BlockSpec((BM, 128), lambda i: (i, 0)),
            pl.BlockSpec((BM, 128), lambda i, _nb=nb: (i + _nb, 0)),
            pl.BlockSpec((BM, 128), lambda i: (i, 0)),
            pl.BlockSpec((D, H1), lambda i: (0, 0)),
            pl.BlockSpec((1, H1), lambda i: (0, 0))],
        out_specs=pl.BlockSpec((BM, 128), lambda i: (i, 0)),
        out_shape=jax.ShapeDtypeStruct((NPAD, 128), jnp.float32),
    )(deg2, deg2, parts, parts, u, W1, b1_row)


def _final_body(d0_ref, d1_ref, q0_ref, q1_ref, v_ref, w_ref, b_ref,
                wfc_ref, bfc_ref, y_ref):
    dis = _dis(d0_ref, d1_ref)
    z = (q0_ref[...] + q1_ref[...] + v_ref[...])[:, :H1]
    o = jnp.maximum(dis * _DOT(z, w_ref[...]) + b_ref[...], 0.0)
    y_ref[...] = _DOT(o, wfc_ref[...]) + bfc_ref[...]


def _tc_final(deg2, parts, v, W2, b2_row, Wfc, bfc_row):
    nb = NPAD // BM
    return pl.pallas_call(
        _final_body,
        grid=(nb,),
        in_specs=_deg_spec(nb) + [
            pl.BlockSpec((BM, 128), lambda i: (i, 0)),
            pl.BlockSpec((BM, 128), lambda i, _nb=nb: (i + _nb, 0)),
            pl.BlockSpec((BM, 128), lambda i: (i, 0)),
            pl.BlockSpec((H1, H2), lambda i: (0, 0)),
            pl.BlockSpec((1, H2), lambda i: (0, 0)),
            pl.BlockSpec((H2, 1), lambda i: (0, 0)),
            pl.BlockSpec((1, 1), lambda i: (0, 0))],
        out_specs=pl.BlockSpec((BM, 1), lambda i: (i, 0)),
        out_shape=jax.ShapeDtypeStruct((NPAD, 1), jnp.float32),
    )(deg2, deg2, parts, parts, v, W2, b2_row, Wfc, bfc_row)


def kernel(x, edge_index, W1, b1, W2, b2, Wfc, bfc):
    x_pad = jnp.pad(x, ((0, NPAD - N), (0, 0)))
    pad_idx = jnp.full((ER * K - E,), NPAD - 1, jnp.int32)
    src2d = jnp.concatenate([edge_index[0], pad_idx]).reshape(ER, K)
    dst2d = jnp.concatenate([edge_index[1], pad_idx]).reshape(ER, K)

    deg2 = _sc_degree(dst2d).reshape(NC * NPAD, 1)
    u = _tc_scale(deg2, x_pad)
    p1 = _sc_aggregate(u, src2d, dst2d)
    v = _tc_layer1(deg2, p1, u, W1, b1.reshape(1, H1))
    p2 = _sc_aggregate(v, src2d, dst2d)
    y = _tc_final(deg2, p2, v, W2, b2.reshape(1, H2), Wfc,
                  bfc.reshape(1, 1))
    return y[:N, 0]


# speedup vs baseline: 1.0086x; 1.0086x over previous
import dataclasses
import functools

import jax
import jax.numpy as jnp
from jax import lax
from jax.experimental import pallas as pl
from jax.experimental.pallas import tpu as pltpu
from jax.experimental.pallas import tpu_sc as plsc

N = 10000
E = 320000
D = 128
H1 = 64
H2 = 32

NC = 2
NS = 16
NW = NC * NS
K = 128

NPAD = 10240
ER = 2560
EPT = ER // NW
APT = NPAD // NS
IB = 16


def _sc_degree(dst2d):
    mesh = plsc.VectorSubcoreMesh(core_axis_name="c", subcore_axis_name="s")

    hr = NPAD // 128

    cp = pltpu.CompilerParams()
    if "needs_layout_passes" in pltpu.CompilerParams.__dataclass_fields__:
        cp = dataclasses.replace(cp, needs_layout_passes=False)

    @functools.partial(
        pl.kernel,
        out_type=jax.ShapeDtypeStruct((NC * hr, 128), jnp.float32),
        mesh=mesh,
        compiler_params=cp,
        scratch_types=[
            pltpu.VMEM((EPT, K), jnp.int32),
            pltpu.VMEM((hr, 128), jnp.float32),
            pltpu.VMEM((hr // 16, 16), jnp.int32),
            pltpu.VMEM_SHARED((hr, 128), jnp.float32),
            pltpu.SemaphoreType.DMA,
        ],
    )
    def deg_kernel(dst_hbm, out_hbm, idx_v, hist_v, idr_v, acc, sem):
        cid = lax.axis_index("c")
        sid = lax.axis_index("s")
        wid = sid * NC + cid

        pltpu.async_copy(
            dst_hbm.at[pl.ds(pl.multiple_of(wid * EPT, 8), EPT)], idx_v, sem)

        @pl.loop(0, hr)
        def _(r):
            @pl.loop(0, 128, step=16)
            def _(c):
                hist_v[r, pl.ds(c, 16)] = jnp.zeros((16,), jnp.float32)

        @pl.loop(0, hr // 16)
        def _(k):
            idr_v[k, :] = lax.iota(jnp.int32, 16) + k * 16

        @pl.when(sid < hr // 8)
        def _():
            pltpu.sync_copy(
                hist_v.at[pl.ds(0, 8)],
                acc.at[pl.ds(pl.multiple_of(sid * 8, 8), 8)])

        pltpu.make_async_copy(
            dst_hbm.at[pl.ds(pl.multiple_of(wid * EPT, 8), EPT)], idx_v,
            sem).wait()
        plsc.subcore_barrier()

        ones16 = jnp.ones((16,), jnp.float32)

        @pl.loop(0, EPT)
        def _(r):
            @pl.loop(0, K, step=16)
            def _(c):
                node = idx_v[r, pl.ds(c, 16)]
                plsc.addupdate_scatter(
                    hist_v,
                    [lax.shift_right_logical(node, 7),
                     lax.bitwise_and(node, 127)],
                    ones16)

        @pl.loop(0, hr // 16)
        def _(k):
            pltpu.sync_copy(
                hist_v.at[pl.ds(pl.multiple_of(k * 16, 8), 16)],
                acc.at[idr_v.at[k]], add=True)

        plsc.subcore_barrier()

        @pl.when(sid < hr // 8)
        def _():
            pltpu.sync_copy(
                acc.at[pl.ds(pl.multiple_of(sid * 8, 8), 8)],
                out_hbm.at[pl.ds(pl.multiple_of(cid * hr + sid * 8, 8), 8)])

    return deg_kernel(dst2d)


def _sc_aggregate(u, src2d, dst2d):
    mesh = plsc.VectorSubcoreMesh(core_axis_name="c", subcore_axis_name="s")

    @functools.partial(
        pl.kernel,
        out_type=jax.ShapeDtypeStruct((NC * NPAD, 128), jnp.float32),
        mesh=mesh,
        scratch_types=[
            pltpu.VMEM((IB, K), jnp.int32),
            pltpu.VMEM((IB, K), jnp.int32),
            pltpu.VMEM((K, 128), jnp.float32),
            pltpu.VMEM((K, 128), jnp.float32),
            pltpu.VMEM((40, 128), jnp.float32),
            pltpu.VMEM_SHARED((NPAD, 128), jnp.float32),
            pltpu.SemaphoreType.DMA,
            pltpu.SemaphoreType.DMA,
            pltpu.SemaphoreType.DMA,
        ],
    )
    def agg_kernel(u_hbm, src_hbm, dst_hbm, out_hbm,
                   src_v, dst_v, buf_a, buf_b, zero_v, acc,
                   sem_i, sem_a0, sem_b0):
        cid = lax.axis_index("c")
        sid = lax.axis_index("s")
        wid = sid * NC + cid

        @pl.loop(0, 40)
        def _(r):
            @pl.loop(0, 128, step=16)
            def _(c):
                zero_v[r, pl.ds(c, 16)] = jnp.zeros((16,), jnp.float32)

        base = pl.multiple_of(sid * APT, 8)

        @pl.loop(0, APT // 40)
        def _(j):
            pltpu.sync_copy(zero_v, acc.at[pl.ds(pl.multiple_of(base + j * 40, 8), 40)])

        ebase = wid * EPT
        plsc.subcore_barrier()

        @pl.loop(0, EPT // IB)
        def _(blk):
            off = pl.multiple_of(ebase + blk * IB, 8)
            pltpu.async_copy(src_hbm.at[pl.ds(off, IB)], src_v, sem_i).wait()
            pltpu.async_copy(dst_hbm.at[pl.ds(off, IB)], dst_v, sem_i).wait()

            pltpu.async_copy(u_hbm.at[src_v.at[0]], buf_a, sem_a0)

            @pl.loop(0, IB, step=2)
            def _(j):
                pltpu.async_copy(u_hbm.at[src_v.at[j + 1]], buf_b, sem_b0)
                pltpu.make_async_copy(
                    u_hbm.at[src_v.at[j]], buf_a, sem_a0).wait()
                pltpu.sync_copy(buf_a, acc.at[dst_v.at[j]], add=True)

                @pl.when(j + 2 < IB)
                def _():
                    pltpu.async_copy(u_hbm.at[src_v.at[j + 2]], buf_a, sem_a0)

                pltpu.make_async_copy(
                    u_hbm.at[src_v.at[j + 1]], buf_b, sem_b0).wait()
                pltpu.sync_copy(buf_b, acc.at[dst_v.at[j + 1]], add=True)

        plsc.subcore_barrier()
        pltpu.sync_copy(acc.at[pl.ds(base, APT)],
                        out_hbm.at[pl.ds(pl.multiple_of(cid * NPAD + base, 8), APT)])

    return agg_kernel(u, src2d, dst2d)


_DOT = functools.partial(
    lax.dot_general,
    dimension_numbers=(((1,), (0,)), ((), ())),
    preferred_element_type=jnp.float32,
    precision=lax.Precision.HIGHEST,
)

BM = 1024


def _dis(d0_ref, d1_ref):
    return lax.rsqrt(d0_ref[...] + d1_ref[...] + 1.0)


def _deg_spec(nb):
    return [pl.BlockSpec((BM, 1), lambda i: (i, 0)),
            pl.BlockSpec((BM, 1), lambda i, _nb=nb: (i + _nb, 0))]


def _scale_body(d0_ref, d1_ref, x_ref, u_ref):
    u_ref[...] = _dis(d0_ref, d1_ref) * x_ref[...]


def _tc_scale(deg2, x_pad):
    nb = NPAD // BM
    return pl.pallas_call(
        _scale_body,
        grid=(nb,),
        in_specs=_deg_spec(nb) + [pl.BlockSpec((BM, D), lambda i: (i, 0))],
        out_specs=pl.BlockSpec((BM, D), lambda i: (i, 0)),
        out_shape=jax.ShapeDtypeStruct((NPAD, D), jnp.float32),
    )(deg2, deg2, x_pad)


def _layer1_body(d0_ref, d1_ref, p0_ref, p1_ref, u_ref, w_ref, b_ref, v_ref):
    dis = _dis(d0_ref, d1_ref)
    z = p0_ref[...] + p1_ref[...] + u_ref[...]
    o = jnp.maximum(dis * _DOT(z, w_ref[...]) + b_ref[...], 0.0)
    v = dis * o
    v_ref[...] = jnp.concatenate(
        [v, jnp.zeros((v.shape[0], 128 - H1), jnp.float32)], axis=1)


def _tc_layer1(deg2, parts, u, W1, b1_row):
    nb = NPAD // BM
    return pl.pallas_call(
        _layer1_body,
        grid=(nb,),
        in_specs=_deg_spec(nb) + [
            pl.BlockSpec((BM, 128), lambda i: (i, 0)),
            pl.BlockSpec((BM, 128), lambda i, _nb=nb: (i + _nb, 0)),
            pl.BlockSpec((BM, 128), lambda i: (i, 0)),
            pl.BlockSpec((D, H1), lambda i: (0, 0)),
            pl.BlockSpec((1, H1), lambda i: (0, 0))],
        out_specs=pl.BlockSpec((BM, 128), lambda i: (i, 0)),
        out_shape=jax.ShapeDtypeStruct((NPAD, 128), jnp.float32),
    )(deg2, deg2, parts, parts, u, W1, b1_row)


def _final_body(d0_ref, d1_ref, q0_ref, q1_ref, v_ref, w_ref, b_ref,
                wfc_ref, bfc_ref, y_ref):
    dis = _dis(d0_ref, d1_ref)
    z = (q0_ref[...] + q1_ref[...] + v_ref[...])[:, :H1]
    o = jnp.maximum(dis * _DOT(z, w_ref[...]) + b_ref[...], 0.0)
    y_ref[...] = _DOT(o, wfc_ref[...]) + bfc_ref[...]


def _tc_final(deg2, parts, v, W2, b2_row, Wfc, bfc_row):
    nb = NPAD // BM
    return pl.pallas_call(
        _final_body,
        grid=(nb,),
        in_specs=_deg_spec(nb) + [
            pl.BlockSpec((BM, 128), lambda i: (i, 0)),
            pl.BlockSpec((BM, 128), lambda i, _nb=nb: (i + _nb, 0)),
            pl.BlockSpec((BM, 128), lambda i: (i, 0)),
            pl.BlockSpec((H1, H2), lambda i: (0, 0)),
            pl.BlockSpec((1, H2), lambda i: (0, 0)),
            pl.BlockSpec((H2, 1), lambda i: (0, 0)),
            pl.BlockSpec((1, 1), lambda i: (0, 0))],
        out_specs=pl.BlockSpec((BM, 1), lambda i: (i, 0)),
        out_shape=jax.ShapeDtypeStruct((NPAD, 1), jnp.float32),
    )(deg2, deg2, parts, parts, v, W2, b2_row, Wfc, bfc_row)


def kernel(x, edge_index, W1, b1, W2, b2, Wfc, bfc):
    x_pad = jnp.pad(x, ((0, NPAD - N), (0, 0)))
    pad_idx = jnp.full((ER * K - E,), NPAD - 1, jnp.int32)
    src2d = jnp.concatenate([edge_index[0], pad_idx]).reshape(ER, K)
    dst2d = jnp.concatenate([edge_index[1], pad_idx]).reshape(ER, K)

    deg2 = _sc_degree(dst2d).reshape(NC * NPAD, 1)
    u = _tc_scale(deg2, x_pad)
    p1 = _sc_aggregate(u, src2d, dst2d)
    v = _tc_layer1(deg2, p1, u, W1, b1.reshape(1, H1))
    p2 = _sc_aggregate(v, src2d, dst2d)
    y = _tc_final(deg2, p2, v, W2, b2.reshape(1, H2), Wfc,
                  bfc.reshape(1, 1))
    return y[:N, 0]


# speedup vs baseline: 1.0181x; 1.0094x over previous
import dataclasses
import functools

import jax
import jax.numpy as jnp
from jax import lax
from jax.experimental import pallas as pl
from jax.experimental.pallas import tpu as pltpu
from jax.experimental.pallas import tpu_sc as plsc

N = 10000
E = 320000
D = 128
H1 = 64
H2 = 32

NC = 2
NS = 16
NW = NC * NS
K = 128

NPAD = 10240
ER = 2560
EPT = ER // NW
APT = NPAD // NS
IB = 16


def _sc_degree(dst2d):
    mesh = plsc.VectorSubcoreMesh(core_axis_name="c", subcore_axis_name="s")

    hr = NPAD // 128

    cp = pltpu.CompilerParams()
    if "needs_layout_passes" in pltpu.CompilerParams.__dataclass_fields__:
        cp = dataclasses.replace(cp, needs_layout_passes=False)

    @functools.partial(
        pl.kernel,
        out_type=jax.ShapeDtypeStruct((NC * hr, 128), jnp.float32),
        mesh=mesh,
        compiler_params=cp,
        scratch_types=[
            pltpu.VMEM((EPT, K), jnp.int32),
            pltpu.VMEM((hr, 128), jnp.float32),
            pltpu.VMEM((hr // 16, 16), jnp.int32),
            pltpu.VMEM_SHARED((hr, 128), jnp.float32),
            pltpu.SemaphoreType.DMA,
        ],
    )
    def deg_kernel(dst_hbm, out_hbm, idx_v, hist_v, idr_v, acc, sem):
        cid = lax.axis_index("c")
        sid = lax.axis_index("s")
        wid = sid * NC + cid

        pltpu.async_copy(
            dst_hbm.at[pl.ds(pl.multiple_of(wid * EPT, 8), EPT)], idx_v, sem)

        @pl.loop(0, hr)
        def _(r):
            @pl.loop(0, 128, step=16)
            def _(c):
                hist_v[r, pl.ds(c, 16)] = jnp.zeros((16,), jnp.float32)

        @pl.loop(0, hr // 16)
        def _(k):
            idr_v[k, :] = lax.iota(jnp.int32, 16) + k * 16

        @pl.when(sid < hr // 8)
        def _():
            pltpu.sync_copy(
                hist_v.at[pl.ds(0, 8)],
                acc.at[pl.ds(pl.multiple_of(sid * 8, 8), 8)])

        pltpu.make_async_copy(
            dst_hbm.at[pl.ds(pl.multiple_of(wid * EPT, 8), EPT)], idx_v,
            sem).wait()
        plsc.subcore_barrier()

        ones16 = jnp.ones((16,), jnp.float32)

        @pl.loop(0, EPT)
        def _(r):
            @pl.loop(0, K, step=16)
            def _(c):
                node = idx_v[r, pl.ds(c, 16)]
                plsc.addupdate_scatter(
                    hist_v,
                    [lax.shift_right_logical(node, 7),
                     lax.bitwise_and(node, 127)],
                    ones16)

        @pl.loop(0, hr // 16)
        def _(k):
            pltpu.sync_copy(
                hist_v.at[pl.ds(pl.multiple_of(k * 16, 8), 16)],
                acc.at[idr_v.at[k]], add=True)

        plsc.subcore_barrier()

        @pl.when(sid < hr // 8)
        def _():
            pltpu.sync_copy(
                acc.at[pl.ds(pl.multiple_of(sid * 8, 8), 8)],
                out_hbm.at[pl.ds(pl.multiple_of(cid * hr + sid * 8, 8), 8)])

    return deg_kernel(dst2d)


def _sc_aggregate(u, src2d, dst2d):
    mesh = plsc.VectorSubcoreMesh(core_axis_name="c", subcore_axis_name="s")

    @functools.partial(
        pl.kernel,
        out_type=jax.ShapeDtypeStruct((NC * NPAD, 128), jnp.float32),
        mesh=mesh,
        scratch_types=[
            pltpu.VMEM((IB, K), jnp.int32),
            pltpu.VMEM((IB, K), jnp.int32),
            pltpu.VMEM((IB, K), jnp.int32),
            pltpu.VMEM((IB, K), jnp.int32),
            pltpu.VMEM((K, 128), jnp.float32),
            pltpu.VMEM((K, 128), jnp.float32),
            pltpu.VMEM((40, 128), jnp.float32),
            pltpu.VMEM_SHARED((NPAD, 128), jnp.float32),
            pltpu.SemaphoreType.DMA,
            pltpu.SemaphoreType.DMA,
            pltpu.SemaphoreType.DMA,
            pltpu.SemaphoreType.DMA,
        ],
    )
    def agg_kernel(u_hbm, src_hbm, dst_hbm, out_hbm,
                   src_va, dst_va, src_vb, dst_vb, buf_a, buf_b, zero_v, acc,
                   sem_ia, sem_ib, sem_a0, sem_b0):
        cid = lax.axis_index("c")
        sid = lax.axis_index("s")
        wid = sid * NC + cid

        @pl.loop(0, 40)
        def _(r):
            @pl.loop(0, 128, step=16)
            def _(c):
                zero_v[r, pl.ds(c, 16)] = jnp.zeros((16,), jnp.float32)

        base = pl.multiple_of(sid * APT, 8)

        @pl.loop(0, APT // 40)
        def _(j):
            pltpu.sync_copy(zero_v, acc.at[pl.ds(pl.multiple_of(base + j * 40, 8), 40)])

        ebase = wid * EPT
        plsc.subcore_barrier()

        pairs = [(src_va, dst_va, sem_ia), (src_vb, dst_vb, sem_ib)]

        def idx_start(blk, pair):
            sv, dv, sem = pair
            off = pl.multiple_of(ebase + blk * IB, 8)
            pltpu.async_copy(src_hbm.at[pl.ds(off, IB)], sv, sem)
            pltpu.async_copy(dst_hbm.at[pl.ds(off, IB)], dv, sem)

        def idx_wait(blk, pair):
            sv, dv, sem = pair
            off = pl.multiple_of(ebase + blk * IB, 8)
            pltpu.make_async_copy(src_hbm.at[pl.ds(off, IB)], sv, sem).wait()
            pltpu.make_async_copy(dst_hbm.at[pl.ds(off, IB)], dv, sem).wait()

        nblk = EPT // IB
        idx_start(0, pairs[0])
        for blk in range(nblk):
            cur = pairs[blk % 2]
            idx_wait(blk, cur)
            if blk + 1 < nblk:
                idx_start(blk + 1, pairs[(blk + 1) % 2])
            src_v, dst_v = cur[0], cur[1]

            pltpu.async_copy(u_hbm.at[src_v.at[0]], buf_a, sem_a0)

            @pl.loop(0, IB, step=2)
            def _(j, src_v=src_v, dst_v=dst_v):
                pltpu.async_copy(u_hbm.at[src_v.at[j + 1]], buf_b, sem_b0)
                pltpu.make_async_copy(
                    u_hbm.at[src_v.at[j]], buf_a, sem_a0).wait()
                pltpu.sync_copy(buf_a, acc.at[dst_v.at[j]], add=True)

                @pl.when(j + 2 < IB)
                def _():
                    pltpu.async_copy(u_hbm.at[src_v.at[j + 2]], buf_a, sem_a0)

                pltpu.make_async_copy(
                    u_hbm.at[src_v.at[j + 1]], buf_b, sem_b0).wait()
                pltpu.sync_copy(buf_b, acc.at[dst_v.at[j + 1]], add=True)

        plsc.subcore_barrier()
        pltpu.sync_copy(acc.at[pl.ds(base, APT)],
                        out_hbm.at[pl.ds(pl.multiple_of(cid * NPAD + base, 8), APT)])

    return agg_kernel(u, src2d, dst2d)


_DOT = functools.partial(
    lax.dot_general,
    dimension_numbers=(((1,), (0,)), ((), ())),
    preferred_element_type=jnp.float32,
    precision=lax.Precision.HIGHEST,
)

BM = 1024


def _dis(d0_ref, d1_ref):
    return lax.rsqrt(d0_ref[...] + d1_ref[...] + 1.0)


def _deg_spec(nb):
    return [pl.BlockSpec((BM, 1), lambda i: (i, 0)),
            pl.BlockSpec((BM, 1), lambda i, _nb=nb: (i + _nb, 0))]


def _scale_body(d0_ref, d1_ref, x_ref, u_ref):
    u_ref[...] = _dis(d0_ref, d1_ref) * x_ref[...]


def _tc_scale(deg2, x_pad):
    nb = NPAD // BM
    return pl.pallas_call(
        _scale_body,
        grid=(nb,),
        in_specs=_deg_spec(nb) + [pl.BlockSpec((BM, D), lambda i: (i, 0))],
        out_specs=pl.BlockSpec((BM, D), lambda i: (i, 0)),
        out_shape=jax.ShapeDtypeStruct((NPAD, D), jnp.float32),
    )(deg2, deg2, x_pad)


def _layer1_body(d0_ref, d1_ref, p0_ref, p1_ref, u_ref, w_ref, b_ref, v_ref):
    dis = _dis(d0_ref, d1_ref)
    z = p0_ref[...] + p1_ref[...] + u_ref[...]
    o = jnp.maximum(dis * _DOT(z, w_ref[...]) + b_ref[...], 0.0)
    v = dis * o
    v_ref[...] = jnp.concatenate(
        [v, jnp.zeros((v.shape[0], 128 - H1), jnp.float32)], axis=1)


def _tc_layer1(deg2, parts, u, W1, b1_row):
    nb = NPAD // BM
    return pl.pallas_call(
        _layer1_body,
        grid=(nb,),
        in_specs=_deg_spec(nb) + [
            pl.BlockSpec((BM, 128), lambda i: (i, 0)),
            pl.BlockSpec((BM, 128), lambda i, _nb=nb: (i + _nb, 0)),
            pl.BlockSpec((BM, 128), lambda i: (i, 0)),
            pl.BlockSpec((D, H1), lambda i: (0, 0)),
            pl.BlockSpec((1, H1), lambda i: (0, 0))],
        out_specs=pl.BlockSpec((BM, 128), lambda i: (i, 0)),
        out_shape=jax.ShapeDtypeStruct((NPAD, 128), jnp.float32),
    )(deg2, deg2, parts, parts, u, W1, b1_row)


def _final_body(d0_ref, d1_ref, q0_ref, q1_ref, v_ref, w_ref, b_ref,
                wfc_ref, bfc_ref, y_ref):
    dis = _dis(d0_ref, d1_ref)
    z = (q0_ref[...] + q1_ref[...] + v_ref[...])[:, :H1]
    o = jnp.maximum(dis * _DOT(z, w_ref[...]) + b_ref[...], 0.0)
    y_ref[...] = _DOT(o, wfc_ref[...]) + bfc_ref[...]


def _tc_final(deg2, parts, v, W2, b2_row, Wfc, bfc_row):
    nb = NPAD // BM
    return pl.pallas_call(
        _final_body,
        grid=(nb,),
        in_specs=_deg_spec(nb) + [
            pl.BlockSpec((BM, 128), lambda i: (i, 0)),
            pl.BlockSpec((BM, 128), lambda i, _nb=nb: (i + _nb, 0)),
            pl.BlockSpec((BM, 128), lambda i: (i, 0)),
            pl.BlockSpec((H1, H2), lambda i: (0, 0)),
            pl.BlockSpec((1, H2), lambda i: (0, 0)),
            pl.BlockSpec((H2, 1), lambda i: (0, 0)),
            pl.BlockSpec((1, 1), lambda i: (0, 0))],
        out_specs=pl.BlockSpec((BM, 1), lambda i: (i, 0)),
        out_shape=jax.ShapeDtypeStruct((NPAD, 1), jnp.float32),
    )(deg2, deg2, parts, parts, v, W2, b2_row, Wfc, bfc_row)


def kernel(x, edge_index, W1, b1, W2, b2, Wfc, bfc):
    x_pad = jnp.pad(x, ((0, NPAD - N), (0, 0)))
    pad_idx = jnp.full((ER * K - E,), NPAD - 1, jnp.int32)
    src2d = jnp.concatenate([edge_index[0], pad_idx]).reshape(ER, K)
    dst2d = jnp.concatenate([edge_index[1], pad_idx]).reshape(ER, K)

    deg2 = _sc_degree(dst2d).reshape(NC * NPAD, 1)
    u = _tc_scale(deg2, x_pad)
    p1 = _sc_aggregate(u, src2d, dst2d)
    v = _tc_layer1(deg2, p1, u, W1, b1.reshape(1, H1))
    p2 = _sc_aggregate(v, src2d, dst2d)
    y = _tc_final(deg2, p2, v, W2, b2.reshape(1, H2), Wfc,
                  bfc.reshape(1, 1))
    return y[:N, 0]
